# Initial kernel scaffold; baseline (speedup 1.0000x reference)
import jax
import jax.numpy as jnp
from jax.experimental import pallas as pl


def kernel(x, channels):
    raise NotImplementedError("write your pallas kernel here")


# speedup vs baseline: 1.0041x; 1.0041x over previous
import functools
import math

import jax
import jax.numpy as jnp
from jax.experimental import pallas as pl
from jax.experimental.pallas import tpu as pltpu

NUM_PATCHES_K = 1024


def _add_enc_kernel(ch_ref, x_ref, out_ref, enc_ref, *, num_patches, d):
    rb = pl.program_id(0)
    b = pl.program_id(1)
    half = d // 2
    quarter = d // 4
    neg_log_base = -math.log(10000.0) / float(quarter)

    @pl.when((rb == 0) & (b == 0))
    def _init_pos_half():
        p = jax.lax.broadcasted_iota(jnp.int32, (num_patches, half), 0).astype(
            jnp.float32
        )
        col = jax.lax.broadcasted_iota(jnp.int32, (num_patches, half), 1)
        jq = (col % quarter).astype(jnp.float32)
        omega = jnp.exp(jq * neg_log_base)
        val = p * omega
        enc_ref[:, half:] = jnp.where(col < quarter, jnp.sin(val), jnp.cos(val))

    @pl.when(b == 0)
    def _init_ch_half():
        ch = ch_ref[rb].astype(jnp.float32)
        col = jax.lax.broadcasted_iota(jnp.int32, (8, half), 1)
        jq = (col % quarter).astype(jnp.float32)
        omega = jnp.exp(jq * neg_log_base)
        val = ch * omega
        row = jnp.where(col < quarter, jnp.sin(val), jnp.cos(val))
        enc_ref[:, :half] = jnp.broadcast_to(row[0:1, :], (num_patches, half))

    out_ref[...] = x_ref[...] + enc_ref[...][None, :, :]


@jax.jit
def kernel(x, channels):
    B, R, D = x.shape
    C = channels.shape[0]
    P = R // C

    grid = (C, B)
    body = functools.partial(_add_enc_kernel, num_patches=P, d=D)
    return pl.pallas_call(
        body,
        grid_spec=pltpu.PrefetchScalarGridSpec(
            num_scalar_prefetch=1,
            grid=grid,
            in_specs=[
                pl.BlockSpec((1, P, D), lambda rb, b, ch: (b, rb, 0)),
            ],
            out_specs=pl.BlockSpec((1, P, D), lambda rb, b, ch: (b, rb, 0)),
            scratch_shapes=[pltpu.VMEM((P, D), jnp.float32)],
        ),
        out_shape=jax.ShapeDtypeStruct((B, R, D), jnp.float32),
    )(channels, x)


# speedup vs baseline: 1.1344x; 1.1297x over previous
import functools
import math

import jax
import jax.numpy as jnp
from jax.experimental import pallas as pl
from jax.experimental.pallas import tpu as pltpu

NUM_PATCHES_K = 1024


def _add_enc_kernel(ch_ref, x_ref, out_ref, enc_ref, *, num_patches, d):
    rb = pl.program_id(0)
    b = pl.program_id(1)
    half = d // 2
    quarter = d // 4
    neg_log_base = -math.log(10000.0) / float(quarter)

    @pl.when((rb == 0) & (b == 0))
    def _init_pos_half():
        p = jax.lax.broadcasted_iota(jnp.int32, (num_patches, half), 0).astype(
            jnp.float32
        )
        col = jax.lax.broadcasted_iota(jnp.int32, (num_patches, half), 1)
        jq = (col % quarter).astype(jnp.float32)
        omega = jnp.exp(jq * neg_log_base)
        val = p * omega
        enc_ref[:, half:] = jnp.where(col < quarter, jnp.sin(val), jnp.cos(val))

    @pl.when(b == 0)
    def _init_ch_half():
        ch = ch_ref[rb].astype(jnp.float32)
        col = jax.lax.broadcasted_iota(jnp.int32, (8, half), 1)
        jq = (col % quarter).astype(jnp.float32)
        omega = jnp.exp(jq * neg_log_base)
        val = ch * omega
        row = jnp.where(col < quarter, jnp.sin(val), jnp.cos(val))
        enc_ref[:, :half] = jnp.broadcast_to(row[0:1, :], (num_patches, half))

    out_ref[...] = x_ref[...] + enc_ref[...][None, :, :]


@jax.jit
def kernel(x, channels):
    B, R, D = x.shape
    C = channels.shape[0]
    P = R // C

    BB = 4
    grid = (C, B // BB)
    body = functools.partial(_add_enc_kernel, num_patches=P, d=D)
    return pl.pallas_call(
        body,
        grid_spec=pltpu.PrefetchScalarGridSpec(
            num_scalar_prefetch=1,
            grid=grid,
            in_specs=[
                pl.BlockSpec((BB, P, D), lambda rb, b, ch: (b, rb, 0)),
            ],
            out_specs=pl.BlockSpec((BB, P, D), lambda rb, b, ch: (b, rb, 0)),
            scratch_shapes=[pltpu.VMEM((P, D), jnp.float32)],
        ),
        out_shape=jax.ShapeDtypeStruct((B, R, D), jnp.float32),
    )(channels, x)
